# baseline (speedup 1.0000x reference)
import functools

import jax
import jax.numpy as jnp
from jax import lax
from jax.experimental import pallas as pl
from jax.experimental.pallas import tpu as pltpu
from jax.experimental.pallas import tpu_sc as plsc

_NC = 2
_NS = 16
_NW = _NC * _NS

_CHUNK = 128
_K = 4
_GROUP = _CHUNK * _K


@functools.partial(jax.jit, static_argnums=(1, 2))
def _lookup(args, total, dim):
    rows_per_w = total // _NW
    n_groups = rows_per_w // _GROUP
    mesh = plsc.VectorSubcoreMesh(core_axis_name="c", subcore_axis_name="s")

    @functools.partial(
        pl.kernel,
        mesh=mesh,
        out_type=jax.ShapeDtypeStruct((total, dim), jnp.float32),
        scratch_types=[
            pltpu.VMEM((_K, _CHUNK), jnp.int32),
            pltpu.VMEM((_GROUP, dim), jnp.float32),
            pltpu.SemaphoreType.DMA,
        ],
        compiler_params=pltpu.CompilerParams(use_tc_tiling_on_sc=False),
    )
    def body(table_hbm, idx_hbm, out_hbm, idx_v, rows_v, sem):
        wid = lax.axis_index("s") * _NC + lax.axis_index("c")
        row0 = wid * rows_per_w

        def group(g, carry):
            gbase = row0 + g * _GROUP
            pltpu.sync_copy(idx_hbm.at[gbase // _GROUP], idx_v)
            copies = [
                pltpu.async_copy(
                    table_hbm.at[idx_v.at[j]],
                    rows_v.at[pl.ds(j * _CHUNK, _CHUNK)],
                    sem,
                )
                for j in range(_K)
            ]
            for c in copies:
                c.wait()
            pltpu.sync_copy(rows_v, out_hbm.at[pl.ds(gbase, _GROUP)])
            return carry

        lax.fori_loop(0, n_groups, group, 0, unroll=False)

    table, idx2d = args
    return body(table, idx2d)


def kernel(table, indices):
    batch, fields = indices.shape
    dim = table.shape[1]
    total = batch * fields
    idx3d = indices.astype(jnp.int32).reshape(total // _GROUP, _K, _CHUNK)
    out = _lookup((table, idx3d), total, dim)
    return out.reshape(batch, fields, dim)


# speedup vs baseline: 1.0296x; 1.0296x over previous
import functools

import jax
import jax.numpy as jnp
from jax import lax
from jax.experimental import pallas as pl
from jax.experimental.pallas import tpu as pltpu
from jax.experimental.pallas import tpu_sc as plsc

_NC = 2
_NS = 16
_NW = _NC * _NS

_CHUNK = 128
_K = 4
_GROUP = _CHUNK * _K
_NB = 2


@functools.partial(jax.jit, static_argnums=(1, 2))
def _lookup(args, total, dim):
    rows_per_w = total // _NW
    n_groups = rows_per_w // _GROUP
    mesh = plsc.VectorSubcoreMesh(core_axis_name="c", subcore_axis_name="s")

    @functools.partial(
        pl.kernel,
        mesh=mesh,
        out_type=jax.ShapeDtypeStruct((total, dim), jnp.float32),
        scratch_types=[
            pltpu.VMEM((n_groups, _K, _CHUNK), jnp.int32),
            pltpu.VMEM((_NB, _GROUP, dim), jnp.float32),
            [pltpu.SemaphoreType.DMA] * _NB,
            [pltpu.SemaphoreType.DMA] * _NB,
        ],
        compiler_params=pltpu.CompilerParams(use_tc_tiling_on_sc=False),
    )
    def body(table_hbm, idx_hbm, out_hbm, idx_v, rows_v, gsems, wsems):
        wid = lax.axis_index("s") * _NC + lax.axis_index("c")
        row0 = wid * rows_per_w
        g0 = row0 // _GROUP

        def fire_gather(g, b):
            return [
                pltpu.async_copy(
                    table_hbm.at[idx_v.at[g, j]],
                    rows_v.at[b].at[pl.ds(j * _CHUNK, _CHUNK)],
                    gsems[b],
                )
                for j in range(_K)
            ]

        def wait_gather(g, b):
            for j in range(_K):
                pltpu.make_async_copy(
                    table_hbm.at[idx_v.at[g, j]],
                    rows_v.at[b].at[pl.ds(j * _CHUNK, _CHUNK)],
                    gsems[b],
                ).wait()

        def fire_write(g, b):
            pltpu.async_copy(
                rows_v.at[b], out_hbm.at[pl.ds(row0 + g * _GROUP, _GROUP)],
                wsems[b],
            )

        def wait_write(g, b):
            pltpu.make_async_copy(
                rows_v.at[b], out_hbm.at[pl.ds(row0 + g * _GROUP, _GROUP)],
                wsems[b],
            ).wait()

        pltpu.sync_copy(idx_hbm.at[pl.ds(g0, n_groups)], idx_v)
        for b in range(_NB):
            fire_gather(b, b)

        def step(g2, carry):
            for b in range(_NB):
                g = g2 * _NB + b
                wait_gather(g, b)
                fire_write(g, b)
                wait_write(g, b)
                fire_gather(g + _NB, b)
            return carry

        lax.fori_loop(0, (n_groups - _NB) // _NB, step, 0, unroll=False)

        for b in range(_NB):
            g = n_groups - _NB + b
            wait_gather(g, b)
            fire_write(g, b)
        for b in range(_NB):
            wait_write(n_groups - _NB + b, b)

    table, idx3d = args
    return body(table, idx3d)


def kernel(table, indices):
    batch, fields = indices.shape
    dim = table.shape[1]
    total = batch * fields
    idx3d = indices.astype(jnp.int32).reshape(total // _GROUP, _K, _CHUNK)
    out = _lookup((table, idx3d), total, dim)
    return out.reshape(batch, fields, dim)
